# baseline (device time: 20358 ns/iter reference)
import jax
import jax.numpy as jnp
from jax import lax
from jax.experimental import pallas as pl
from jax.experimental.pallas import tpu as pltpu

N_Y = 4
MESH = pl.DeviceIdType.MESH


def kernel(x):
    _, m, n_total = x.shape
    n_chunk = n_total // N_Y
    half = n_chunk // 2
    pm = m // 2
    rh = pm // 2
    x = x.reshape(m, n_total)
    bf16 = jnp.bfloat16
    f32 = jnp.float32

    def body(x_ref, out_ref, xh, comb1, comb2, outh, rb, rc, rd, re,
             rz, rx, ssems, sz_sem, sxa_sem, sxb_sem, rb_sems, rc_sems,
             rd_sems, re_sem, rz_sem, rxa_sem, rxb_sem):
        p = lax.axis_index("x")
        i = lax.axis_index("y")
        z = lax.axis_index("z")
        zh = lax.rem(z, 2)
        zmate = z + 1 - 2 * zh

        barrier = pltpu.get_barrier_semaphore()

        def sig(dev):
            pl.semaphore_signal(barrier, inc=1, device_id=dev,
                                device_id_type=MESH)

        @pl.when(jnp.logical_or(i == 0, i == 3))
        def _():
            sig((p, 1, z)); sig((p, 2, z))
            sig((p, i, zmate)); sig((1 - p, i, z))

        @pl.when(i == 1)
        def _():
            sig((p, 0, z)); sig((p, 2, z))
            sig((p, i, zmate)); sig((1 - p, i, z))

        @pl.when(i == 2)
        def _():
            sig((p, 1, z)); sig((p, 3, z))
            sig((p, i, zmate)); sig((1 - p, i, z))

        for c in range(N_Y):
            xh[c] = x_ref[pl.ds(zh * pm, pm),
                          pl.ds(c * n_chunk + p * half, half)].astype(bf16)

        def rows(h):
            return pl.ds(h * rh, rh)

        def rsend(src, dst, ssem, rsem, y_t):
            r = pltpu.make_async_remote_copy(
                src_ref=src, dst_ref=dst, send_sem=ssem, recv_sem=rsem,
                device_id=(p, y_t, z), device_id_type=MESH)
            r.start()
            return r

        def dsend(src, dst, ssem, rsem, dev):
            r = pltpu.make_async_remote_copy(
                src_ref=src, dst_ref=dst, send_sem=ssem, recv_sem=rsem,
                device_id=dev, device_id_type=MESH)
            r.start()
            return r

        def rwait(dst, rsem):
            r = pltpu.make_async_remote_copy(
                src_ref=dst, dst_ref=dst, send_sem=sz_sem,
                recv_sem=rsem, device_id=(p, i, z), device_id_type=MESH)
            r.wait_recv()

        def edge(nbr, c_far, c_near, c_own, me):
            rs = []
            order = [(c_far, 0, 0), (c_near, 1, 0), (c_far, 0, 1),
                     (c_near, 1, 1), (c_own, 2, 0), (c_own, 2, 1)]
            for k, (c, slot, h) in enumerate(order):
                rs.append(rsend(xh.at[c, rows(h), :], rb.at[slot, rows(h), :],
                                ssems.at[k], rb_sems.at[k], nbr))
            rwait(re, re_sem)
            for h in range(2):
                rwait(rc.at[rows(h), :], rc_sems.at[h])
                outh[rows(h), :] = (xh[me, rows(h), :] + re[rows(h), :]
                                    + rc[rows(h), :])
            for r in rs:
                r.wait_send()

        def middle(near_edge, far_edge, near_mid, c_far, c_near, me):
            rs = [rsend(xh.at[near_edge, :, :], re, ssems.at[0], re_sem,
                        near_edge)]
            for h in range(2):
                rwait(rb.at[0, rows(h), :], rb_sems.at[2 * h])
                comb2[rows(h), :] = rb[0, rows(h), :] + xh[c_far, rows(h), :]
                rs.append(rsend(comb2.at[rows(h), :], rc.at[rows(h), :],
                                ssems.at[1 + 2 * h], rc_sems.at[h], far_edge))
                rwait(rb.at[1, rows(h), :], rb_sems.at[2 * h + 1])
                comb1[rows(h), :] = rb[1, rows(h), :] + xh[c_near, rows(h), :]
                rs.append(rsend(comb1.at[rows(h), :], rd.at[rows(h), :],
                                ssems.at[2 + 2 * h], rd_sems.at[h], near_mid))
            for h in range(2):
                rwait(rb.at[2, rows(h), :], rb_sems.at[4 + h])
                rwait(rd.at[rows(h), :], rd_sems.at[h])
                outh[rows(h), :] = (xh[me, rows(h), :] + rb[2, rows(h), :]
                                    + rd[rows(h), :])
            for r in rs:
                r.wait_send()

        @pl.when(i == 0)
        def _():
            pl.semaphore_wait(barrier, 3)
            edge(1, 3, 2, 1, 0)

        @pl.when(i == 1)
        def _():
            pl.semaphore_wait(barrier, 5)
            middle(0, 3, 2, 3, 2, 1)

        @pl.when(i == 2)
        def _():
            pl.semaphore_wait(barrier, 5)
            middle(3, 0, 1, 0, 1, 2)

        @pl.when(i == 3)
        def _():
            pl.semaphore_wait(barrier, 3)
            edge(2, 0, 1, 2, 3)

        r_z = dsend(outh, rz, sz_sem, rz_sem, (p, i, zmate))
        r_xa = dsend(outh, rx.at[pl.ds(zh * pm, pm), :], sxa_sem, rxa_sem,
                     (1 - p, i, z))
        rwait(rz, rz_sem)
        r_xb = dsend(rz, rx.at[pl.ds((1 - zh) * pm, pm), :], sxb_sem,
                     rxb_sem, (1 - p, i, z))
        out_ref[pl.ds(zh * pm, pm), pl.ds(p * half, half)] = (
            outh[...].astype(f32))
        out_ref[pl.ds((1 - zh) * pm, pm), pl.ds(p * half, half)] = (
            rz[...].astype(f32))
        rwait(rx.at[pl.ds(zh * pm, pm), :], rxa_sem)
        rwait(rx.at[pl.ds((1 - zh) * pm, pm), :], rxb_sem)
        out_ref[:, pl.ds((1 - p) * half, half)] = rx[...].astype(f32)
        r_z.wait_send()
        r_xa.wait_send()
        r_xb.wait_send()

    return pl.pallas_call(
        body,
        out_shape=jax.ShapeDtypeStruct((m, n_chunk), f32),
        in_specs=[pl.BlockSpec(memory_space=pltpu.VMEM)],
        out_specs=pl.BlockSpec(memory_space=pltpu.VMEM),
        scratch_shapes=[
            pltpu.VMEM((N_Y, pm, half), bf16),
            pltpu.VMEM((pm, half), bf16),
            pltpu.VMEM((pm, half), bf16),
            pltpu.VMEM((pm, half), bf16),
            pltpu.VMEM((3, pm, half), bf16),
            pltpu.VMEM((pm, half), bf16),
            pltpu.VMEM((pm, half), bf16),
            pltpu.VMEM((pm, half), bf16),
            pltpu.VMEM((pm, half), bf16),
            pltpu.VMEM((m, half), bf16),
            pltpu.SemaphoreType.DMA((6,)),
            pltpu.SemaphoreType.DMA,
            pltpu.SemaphoreType.DMA,
            pltpu.SemaphoreType.DMA,
            pltpu.SemaphoreType.DMA((6,)),
            pltpu.SemaphoreType.DMA((2,)),
            pltpu.SemaphoreType.DMA((2,)),
            pltpu.SemaphoreType.DMA,
            pltpu.SemaphoreType.DMA,
            pltpu.SemaphoreType.DMA,
            pltpu.SemaphoreType.DMA,
        ],
        compiler_params=pltpu.CompilerParams(collective_id=0),
    )(x)


# device time: 19762 ns/iter; 1.0302x vs baseline; 1.0302x over previous
import jax
import jax.numpy as jnp
from jax import lax
from jax.experimental import pallas as pl
from jax.experimental.pallas import tpu as pltpu

N_Y = 4
MESH = pl.DeviceIdType.MESH


def kernel(x):
    _, m, n_total = x.shape
    n_chunk = n_total // N_Y
    half = n_chunk // 2
    rh = m // 2
    x = x.reshape(m, n_total)
    bf16 = jnp.bfloat16
    f32 = jnp.float32

    def body(x_ref, out_ref, xh, comb1, comb2, outh, rb, rc, rd, re, rx,
             ssems, sx_sems, rb_sems, rc_sems, rd_sems, re_sem, rx_sems):
        p = lax.axis_index("x")
        i = lax.axis_index("y")
        z = lax.axis_index("z")

        barrier = pltpu.get_barrier_semaphore()

        def sig_y(y_t):
            pl.semaphore_signal(barrier, inc=1, device_id=(p, y_t, z),
                                device_id_type=MESH)

        def sig_x():
            pl.semaphore_signal(barrier, inc=1, device_id=(1 - p, i, z),
                                device_id_type=MESH)

        @pl.when(jnp.logical_or(i == 0, i == 3))
        def _():
            sig_y(1); sig_y(2); sig_x()

        @pl.when(i == 1)
        def _():
            sig_y(0); sig_y(2); sig_x()

        @pl.when(i == 2)
        def _():
            sig_y(1); sig_y(3); sig_x()

        for c in range(N_Y):
            xh[c] = x_ref[:, pl.ds(c * n_chunk + p * half, half)].astype(bf16)

        def rows(h):
            return pl.ds(h * rh, rh)

        def rsend(src, dst, ssem, rsem, y_t):
            r = pltpu.make_async_remote_copy(
                src_ref=src, dst_ref=dst, send_sem=ssem, recv_sem=rsem,
                device_id=(p, y_t, z), device_id_type=MESH)
            r.start()
            return r

        def xsend(src, dst, ssem, rsem):
            r = pltpu.make_async_remote_copy(
                src_ref=src, dst_ref=dst, send_sem=ssem, recv_sem=rsem,
                device_id=(1 - p, i, z), device_id_type=MESH)
            r.start()
            return r

        def rwait(dst, rsem):
            r = pltpu.make_async_remote_copy(
                src_ref=dst, dst_ref=dst, send_sem=sx_sems.at[0],
                recv_sem=rsem, device_id=(p, i, z), device_id_type=MESH)
            r.wait_recv()

        def edge(nbr, c_far, c_near, c_own, me):
            rs = []
            order = [(c_far, 0, 0), (c_near, 1, 0), (c_far, 0, 1),
                     (c_near, 1, 1), (c_own, 2, 0), (c_own, 2, 1)]
            for k, (c, slot, h) in enumerate(order):
                rs.append(rsend(xh.at[c, rows(h), :], rb.at[slot, rows(h), :],
                                ssems.at[k], rb_sems.at[k], nbr))
            rwait(re, re_sem)
            for h in range(2):
                rwait(rc.at[rows(h), :], rc_sems.at[h])
                outh[rows(h), :] = (xh[me, rows(h), :] + re[rows(h), :]
                                    + rc[rows(h), :])
                rs.append(xsend(outh.at[rows(h), :], rx.at[rows(h), :],
                                sx_sems.at[h], rx_sems.at[h]))
            for r in rs:
                r.wait_send()

        def middle(near_edge, far_edge, near_mid, c_far, c_near, me):
            rs = [rsend(xh.at[near_edge, :, :], re, ssems.at[0], re_sem,
                        near_edge)]
            for h in range(2):
                rwait(rb.at[0, rows(h), :], rb_sems.at[2 * h])
                comb2[rows(h), :] = rb[0, rows(h), :] + xh[c_far, rows(h), :]
                rs.append(rsend(comb2.at[rows(h), :], rc.at[rows(h), :],
                                ssems.at[1 + 2 * h], rc_sems.at[h], far_edge))
                rwait(rb.at[1, rows(h), :], rb_sems.at[2 * h + 1])
                comb1[rows(h), :] = rb[1, rows(h), :] + xh[c_near, rows(h), :]
                rs.append(rsend(comb1.at[rows(h), :], rd.at[rows(h), :],
                                ssems.at[2 + 2 * h], rd_sems.at[h], near_mid))
            for h in range(2):
                rwait(rb.at[2, rows(h), :], rb_sems.at[4 + h])
                rwait(rd.at[rows(h), :], rd_sems.at[h])
                outh[rows(h), :] = (xh[me, rows(h), :] + rb[2, rows(h), :]
                                    + rd[rows(h), :])
                rs.append(xsend(outh.at[rows(h), :], rx.at[rows(h), :],
                                sx_sems.at[h], rx_sems.at[h]))
            for r in rs:
                r.wait_send()

        @pl.when(i == 0)
        def _():
            pl.semaphore_wait(barrier, 2)
            edge(1, 3, 2, 1, 0)

        @pl.when(i == 1)
        def _():
            pl.semaphore_wait(barrier, 4)
            middle(0, 3, 2, 3, 2, 1)

        @pl.when(i == 2)
        def _():
            pl.semaphore_wait(barrier, 4)
            middle(3, 0, 1, 0, 1, 2)

        @pl.when(i == 3)
        def _():
            pl.semaphore_wait(barrier, 2)
            edge(2, 0, 1, 2, 3)

        for h in range(2):
            rwait(rx.at[rows(h), :], rx_sems.at[h])
        out_ref[:, pl.ds(p * half, half)] = outh[...].astype(f32)
        out_ref[:, pl.ds((1 - p) * half, half)] = rx[...].astype(f32)

    return pl.pallas_call(
        body,
        out_shape=jax.ShapeDtypeStruct((m, n_chunk), f32),
        in_specs=[pl.BlockSpec(memory_space=pltpu.VMEM)],
        out_specs=pl.BlockSpec(memory_space=pltpu.VMEM),
        scratch_shapes=[
            pltpu.VMEM((N_Y, m, half), bf16),
            pltpu.VMEM((m, half), bf16),
            pltpu.VMEM((m, half), bf16),
            pltpu.VMEM((m, half), bf16),
            pltpu.VMEM((3, m, half), bf16),
            pltpu.VMEM((m, half), bf16),
            pltpu.VMEM((m, half), bf16),
            pltpu.VMEM((m, half), bf16),
            pltpu.VMEM((m, half), bf16),
            pltpu.SemaphoreType.DMA((6,)),
            pltpu.SemaphoreType.DMA((2,)),
            pltpu.SemaphoreType.DMA((6,)),
            pltpu.SemaphoreType.DMA((2,)),
            pltpu.SemaphoreType.DMA((2,)),
            pltpu.SemaphoreType.DMA,
            pltpu.SemaphoreType.DMA((2,)),
        ],
        compiler_params=pltpu.CompilerParams(collective_id=0),
    )(x)
